# Initial kernel scaffold; baseline (speedup 1.0000x reference)
import jax
import jax.numpy as jnp
from jax.experimental import pallas as pl


def kernel(x, edge_index, W_self, b_self, W_neigh, b_neigh):
    raise NotImplementedError("write your pallas kernel here")


# speedup vs baseline: 5.6893x; 5.6893x over previous
import functools

import jax
import jax.numpy as jnp
from jax import lax
from jax.experimental import pallas as pl
from jax.experimental.pallas import tpu as pltpu
from jax.experimental.pallas import tpu_sc as plsc

_NC = 2
_NS = 16


@functools.lru_cache(maxsize=None)
def _make_sc_agg(N, C, E):
    NW = _NC * _NS
    EPW = E // NW
    assert E % NW == 0
    K = 80
    assert EPW % K == 0 and K % 16 == 0
    STEPS = EPW // K
    ROWS_PER = 640
    STRIDE = 632
    assert (_NS - 1) * STRIDE + ROWS_PER >= N and N % 8 == 0 and N % 16 == 0

    mesh = plsc.VectorSubcoreMesh(core_axis_name="c", subcore_axis_name="s")

    @functools.partial(
        pl.kernel,
        mesh=mesh,
        compiler_params=pltpu.CompilerParams(needs_layout_passes=False),
        out_type=[
            jax.ShapeDtypeStruct((_NC * N, C), jnp.float32),
            jax.ShapeDtypeStruct((_NC * N,), jnp.float32),
        ],
        scratch_types=[
            pltpu.VMEM((K,), jnp.int32),
            pltpu.VMEM((K,), jnp.int32),
            pltpu.VMEM((K, C), jnp.float32),
            pltpu.VMEM((N,), jnp.float32),
            pltpu.VMEM((ROWS_PER,), jnp.float32),
            pltpu.VMEM((ROWS_PER,), jnp.float32),
            pltpu.VMEM_SHARED((N, C), jnp.float32),
            pltpu.VMEM_SHARED((_NS * N,), jnp.float32),
            pltpu.SemaphoreType.DMA,
        ],
    )
    def sc_agg(src_hbm, dst_hbm, x_hbm, z_hbm, agg_out, deg_out,
               dsti, srci, rows, cnt, acc, tmp, aggsh, cntsh, sem):
        cid = lax.axis_index("c")
        sid = lax.axis_index("s")
        w = cid * _NS + sid

        ones16 = jnp.full((16,), 1.0, dtype=jnp.float32)
        z16 = jnp.zeros((16,), jnp.float32)

        def zero_cnt(i, carry):
            cnt[pl.ds(i * 16, 16)] = z16
            return carry

        lax.fori_loop(0, N // 16, zero_cnt, 0)

        r0 = pl.multiple_of(jnp.minimum(sid * STRIDE, N - ROWS_PER), 8)
        o0 = pl.multiple_of(cid * N + r0, 8)

        pltpu.sync_copy(z_hbm.at[pl.ds(r0, ROWS_PER)],
                        aggsh.at[pl.ds(r0, ROWS_PER)])
        plsc.subcore_barrier()

        def step(t, carry):
            base = pl.multiple_of(w * EPW + t * K, 8)
            pltpu.sync_copy(dst_hbm.at[pl.ds(base, K)], dsti)
            pltpu.sync_copy(src_hbm.at[pl.ds(base, K)], srci)
            gather = pltpu.async_copy(x_hbm.at[dsti], rows, sem)
            for j in range(K // 16):
                idxv = srci[pl.ds(j * 16, 16)]
                plsc.addupdate_scatter(cnt, [idxv], ones16)
            gather.wait()
            pltpu.sync_copy(rows, aggsh.at[srci], add=True)
            return carry

        lax.fori_loop(0, STEPS, step, 0)
        plsc.subcore_barrier()

        pltpu.sync_copy(aggsh.at[pl.ds(r0, ROWS_PER)],
                        agg_out.at[pl.ds(o0, ROWS_PER)])
        s0 = pl.multiple_of(sid * N, 8)
        pltpu.sync_copy(cnt, cntsh.at[pl.ds(s0, N)])
        plsc.subcore_barrier()

        def zero_acc(i, carry):
            acc[pl.ds(i * 16, 16)] = z16
            return carry

        lax.fori_loop(0, ROWS_PER // 16, zero_acc, 0)

        def combine(r, carry):
            pltpu.sync_copy(cntsh.at[pl.ds(pl.multiple_of(r * N, 8) + r0,
                                           ROWS_PER)], tmp)

            def addv(i, c2):
                acc[pl.ds(i * 16, 16)] = (acc[pl.ds(i * 16, 16)]
                                          + tmp[pl.ds(i * 16, 16)])
                return c2

            lax.fori_loop(0, ROWS_PER // 16, addv, 0)
            return carry

        lax.fori_loop(0, _NS, combine, 0)
        pltpu.sync_copy(acc, deg_out.at[pl.ds(o0, ROWS_PER)])

    return sc_agg


@functools.lru_cache(maxsize=None)
def _make_tc_dense(N, C, D):
    R = 1000
    assert N % R == 0
    G = N // R

    def body(x_ref, a_ref, d_ref, ws_ref, bs_ref, wn_ref, bn_ref, o_ref):
        x = x_ref[...]
        agg = a_ref[0] + a_ref[1]
        deg = d_ref[0] + d_ref[1] + 1.0
        y = (agg + x) * (1.0 / deg)
        h = (jnp.dot(x, ws_ref[...], preferred_element_type=jnp.float32)
             + bs_ref[...]
             + jnp.dot(y, wn_ref[...], preferred_element_type=jnp.float32)
             + bn_ref[...])
        o_ref[...] = 0.5 * h * (1.0 + lax.erf(h * 0.7071067811865476))

    return pl.pallas_call(
        body,
        grid=(G,),
        in_specs=[
            pl.BlockSpec((R, C), lambda i: (i, 0)),
            pl.BlockSpec((_NC, R, C), lambda i: (0, i, 0)),
            pl.BlockSpec((_NC, R, 1), lambda i: (0, i, 0)),
            pl.BlockSpec((C, D), lambda i: (0, 0)),
            pl.BlockSpec((1, D), lambda i: (0, 0)),
            pl.BlockSpec((C, D), lambda i: (0, 0)),
            pl.BlockSpec((1, D), lambda i: (0, 0)),
        ],
        out_specs=pl.BlockSpec((R, D), lambda i: (i, 0)),
        out_shape=jax.ShapeDtypeStruct((N, D), jnp.float32),
    )


def kernel(x, edge_index, W_self, b_self, W_neigh, b_neigh):
    B, N, C = x.shape
    D = W_self.shape[1]
    E = edge_index.shape[1]
    x2d = x.reshape(N, C)
    src = edge_index[0]
    dst = edge_index[1]
    z = jnp.zeros((N, C), jnp.float32)

    agg_p, deg_p = _make_sc_agg(N, C, E)(src, dst, x2d, z)
    out = _make_tc_dense(N, C, D)(
        x2d, agg_p.reshape(_NC, N, C), deg_p.reshape(_NC, N, 1),
        W_self, b_self.reshape(1, D), W_neigh, b_neigh.reshape(1, D))
    return out.reshape(B, N, D)


# speedup vs baseline: 7.4129x; 1.3030x over previous
import functools

import jax
import jax.numpy as jnp
from jax import lax
from jax.experimental import pallas as pl
from jax.experimental.pallas import tpu as pltpu
from jax.experimental.pallas import tpu_sc as plsc

_NC = 2
_NS = 16


@functools.lru_cache(maxsize=None)
def _make_sc_agg(N, C, E):
    NW = _NC * _NS
    EPW = E // NW
    assert E % NW == 0
    K = 80
    assert EPW % K == 0 and K % 16 == 0
    STEPS = EPW // K
    ROWS_PER = 640
    STRIDE = 632
    assert (_NS - 1) * STRIDE + ROWS_PER >= N and N % 8 == 0 and N % 16 == 0

    mesh = plsc.VectorSubcoreMesh(core_axis_name="c", subcore_axis_name="s")

    @functools.partial(
        pl.kernel,
        mesh=mesh,
        compiler_params=pltpu.CompilerParams(needs_layout_passes=False),
        out_type=[
            jax.ShapeDtypeStruct((_NC * N, C), jnp.float32),
            jax.ShapeDtypeStruct((_NC * N,), jnp.float32),
        ],
        scratch_types=[
            pltpu.VMEM((K,), jnp.int32),
            pltpu.VMEM((K,), jnp.int32),
            pltpu.VMEM((K,), jnp.int32),
            pltpu.VMEM((K,), jnp.int32),
            pltpu.VMEM((K, C), jnp.float32),
            pltpu.VMEM((K, C), jnp.float32),
            pltpu.VMEM((N,), jnp.float32),
            pltpu.VMEM((ROWS_PER,), jnp.float32),
            pltpu.VMEM((ROWS_PER,), jnp.float32),
            pltpu.VMEM_SHARED((N, C), jnp.float32),
            pltpu.VMEM_SHARED((_NS * N,), jnp.float32),
            pltpu.SemaphoreType.DMA,
            pltpu.SemaphoreType.DMA,
            pltpu.SemaphoreType.DMA,
            pltpu.SemaphoreType.DMA,
        ],
    )
    def sc_agg(src_hbm, dst_hbm, x_hbm, z_hbm, agg_out, deg_out,
               dstiA, srciA, dstiB, srciB, rowsA, rowsB,
               cnt, acc, tmp, aggsh, cntsh, gsemA, gsemB, ssemA, ssemB):
        cid = lax.axis_index("c")
        sid = lax.axis_index("s")
        w = cid * _NS + sid

        ones16 = jnp.full((16,), 1.0, dtype=jnp.float32)
        z16 = jnp.zeros((16,), jnp.float32)

        def zero_cnt(i, carry):
            cnt[pl.ds(i * 16, 16)] = z16
            return carry

        lax.fori_loop(0, N // 16, zero_cnt, 0)

        r0 = pl.multiple_of(jnp.minimum(sid * STRIDE, N - ROWS_PER), 8)
        o0 = pl.multiple_of(cid * N + r0, 8)

        pltpu.sync_copy(z_hbm.at[pl.ds(r0, ROWS_PER)],
                        aggsh.at[pl.ds(r0, ROWS_PER)])
        plsc.subcore_barrier()

        def hist(srci):
            for j in range(K // 16):
                idxv = srci[pl.ds(j * 16, 16)]
                plsc.addupdate_scatter(cnt, [idxv], ones16)

        def pair(t, carry):
            baseA = pl.multiple_of(w * EPW + (2 * t) * K, 8)
            baseB = pl.multiple_of(baseA + K, 8)
            pltpu.sync_copy(dst_hbm.at[pl.ds(baseA, K)], dstiA)
            pltpu.sync_copy(src_hbm.at[pl.ds(baseA, K)], srciA)
            gA = pltpu.async_copy(x_hbm.at[dstiA], rowsA, gsemA)
            pltpu.sync_copy(dst_hbm.at[pl.ds(baseB, K)], dstiB)
            pltpu.sync_copy(src_hbm.at[pl.ds(baseB, K)], srciB)
            gB = pltpu.async_copy(x_hbm.at[dstiB], rowsB, gsemB)
            hist(srciA)
            hist(srciB)
            gA.wait()
            sA = pltpu.async_copy(rowsA, aggsh.at[srciA], ssemA, add=True)
            gB.wait()
            sB = pltpu.async_copy(rowsB, aggsh.at[srciB], ssemB, add=True)
            sA.wait()
            sB.wait()
            return carry

        PAIRS = STEPS // 2
        lax.fori_loop(0, PAIRS, pair, 0)
        for t in range(2 * PAIRS, STEPS):
            base = pl.multiple_of(w * EPW + t * K, 8)
            pltpu.sync_copy(dst_hbm.at[pl.ds(base, K)], dstiA)
            pltpu.sync_copy(src_hbm.at[pl.ds(base, K)], srciA)
            pltpu.async_copy(x_hbm.at[dstiA], rowsA, gsemA).wait()
            hist(srciA)
            pltpu.sync_copy(rowsA, aggsh.at[srciA], add=True)
        plsc.subcore_barrier()

        pltpu.sync_copy(aggsh.at[pl.ds(r0, ROWS_PER)],
                        agg_out.at[pl.ds(o0, ROWS_PER)])
        s0 = pl.multiple_of(sid * N, 8)
        pltpu.sync_copy(cnt, cntsh.at[pl.ds(s0, N)])
        plsc.subcore_barrier()

        def zero_acc(i, carry):
            acc[pl.ds(i * 16, 16)] = z16
            return carry

        lax.fori_loop(0, ROWS_PER // 16, zero_acc, 0)

        def combine(r, carry):
            pltpu.sync_copy(cntsh.at[pl.ds(pl.multiple_of(r * N, 8) + r0,
                                           ROWS_PER)], tmp)

            def addv(i, c2):
                acc[pl.ds(i * 16, 16)] = (acc[pl.ds(i * 16, 16)]
                                          + tmp[pl.ds(i * 16, 16)])
                return c2

            lax.fori_loop(0, ROWS_PER // 16, addv, 0)
            return carry

        lax.fori_loop(0, _NS, combine, 0)
        pltpu.sync_copy(acc, deg_out.at[pl.ds(o0, ROWS_PER)])

    return sc_agg


@functools.lru_cache(maxsize=None)
def _make_tc_dense(N, C, D):
    R = 1000
    assert N % R == 0
    G = N // R

    def body(x_ref, a_ref, d_ref, ws_ref, bs_ref, wn_ref, bn_ref, o_ref):
        x = x_ref[...]
        agg = a_ref[0] + a_ref[1]
        deg = d_ref[0] + d_ref[1] + 1.0
        y = (agg + x) * (1.0 / deg)
        h = (jnp.dot(x, ws_ref[...], preferred_element_type=jnp.float32)
             + bs_ref[...]
             + jnp.dot(y, wn_ref[...], preferred_element_type=jnp.float32)
             + bn_ref[...])
        o_ref[...] = 0.5 * h * (1.0 + lax.erf(h * 0.7071067811865476))

    return pl.pallas_call(
        body,
        grid=(G,),
        in_specs=[
            pl.BlockSpec((R, C), lambda i: (i, 0)),
            pl.BlockSpec((_NC, R, C), lambda i: (0, i, 0)),
            pl.BlockSpec((_NC, R, 1), lambda i: (0, i, 0)),
            pl.BlockSpec((C, D), lambda i: (0, 0)),
            pl.BlockSpec((1, D), lambda i: (0, 0)),
            pl.BlockSpec((C, D), lambda i: (0, 0)),
            pl.BlockSpec((1, D), lambda i: (0, 0)),
        ],
        out_specs=pl.BlockSpec((R, D), lambda i: (i, 0)),
        out_shape=jax.ShapeDtypeStruct((N, D), jnp.float32),
    )


def kernel(x, edge_index, W_self, b_self, W_neigh, b_neigh):
    B, N, C = x.shape
    D = W_self.shape[1]
    E = edge_index.shape[1]
    x2d = x.reshape(N, C)
    src = edge_index[0]
    dst = edge_index[1]
    z = jnp.zeros((N, C), jnp.float32)

    agg_p, deg_p = _make_sc_agg(N, C, E)(src, dst, x2d, z)
    out = _make_tc_dense(N, C, D)(
        x2d, agg_p.reshape(_NC, N, C), deg_p.reshape(_NC, N, 1),
        W_self, b_self.reshape(1, D), W_neigh, b_neigh.reshape(1, D))
    return out.reshape(B, N, D)


# speedup vs baseline: 8.5201x; 1.1494x over previous
import functools

import jax
import jax.numpy as jnp
from jax import lax
from jax.experimental import pallas as pl
from jax.experimental.pallas import tpu as pltpu
from jax.experimental.pallas import tpu_sc as plsc

_NC = 2
_NS = 16
_K = 40


@functools.lru_cache(maxsize=None)
def _make_sc_agg(N, C, E):
    NW = _NC * _NS
    EPW = E // NW
    assert E % NW == 0
    K = _K
    assert EPW % K == 0 and K % 8 == 0
    STEPS = EPW // K
    ROWS_PER = 640
    STRIDE = 632
    assert (_NS - 1) * STRIDE + ROWS_PER >= N and N % 8 == 0 and N % 16 == 0

    mesh = plsc.VectorSubcoreMesh(core_axis_name="c", subcore_axis_name="s")

    @functools.partial(
        pl.kernel,
        mesh=mesh,
        compiler_params=pltpu.CompilerParams(needs_layout_passes=False),
        out_type=[
            jax.ShapeDtypeStruct((_NC * N, C), jnp.float32),
            jax.ShapeDtypeStruct((_NC * N,), jnp.float32),
        ],
        scratch_types=[
            pltpu.VMEM((8, K), jnp.int32),
            pltpu.VMEM((8, K), jnp.int32),
            pltpu.VMEM((K,), jnp.int32),
            pltpu.VMEM((K,), jnp.int32),
            pltpu.VMEM((8 * K,), jnp.int32),
            pltpu.VMEM((K, C), jnp.float32),
            pltpu.VMEM((K, C), jnp.float32),
            pltpu.VMEM((K, C), jnp.float32),
            pltpu.VMEM((K, C), jnp.float32),
            pltpu.VMEM((N,), jnp.float32),
            pltpu.VMEM((ROWS_PER,), jnp.float32),
            pltpu.VMEM((ROWS_PER,), jnp.float32),
            pltpu.VMEM_SHARED((N, C), jnp.float32),
            pltpu.VMEM_SHARED((_NS * N,), jnp.float32),
            pltpu.SemaphoreType.DMA,
            pltpu.SemaphoreType.DMA,
            pltpu.SemaphoreType.DMA,
            pltpu.SemaphoreType.DMA,
            pltpu.SemaphoreType.DMA,
            pltpu.SemaphoreType.DMA,
            pltpu.SemaphoreType.DMA,
            pltpu.SemaphoreType.DMA,
        ],
    )
    def sc_agg(src_hbm, dst_hbm, srcf_hbm, dstf_hbm, x_hbm, z_hbm,
               agg_out, deg_out,
               dstb, srcb, dsti1, srci1, srcl, rows0, rows1, rows2, rows3,
               cnt, acc, tmp, aggsh, cntsh,
               gsem0, gsem1, gsem2, gsem3, ssem0, ssem1, ssem2, ssem3):
        cid = lax.axis_index("c")
        sid = lax.axis_index("s")
        w = cid * _NS + sid

        ones16 = jnp.full((16,), 1.0, dtype=jnp.float32)
        z16 = jnp.zeros((16,), jnp.float32)

        def zero_cnt(i, carry):
            cnt[pl.ds(i * 16, 16)] = z16
            return carry

        lax.fori_loop(0, N // 16, zero_cnt, 0)

        r0 = pl.multiple_of(jnp.minimum(sid * STRIDE, N - ROWS_PER), 8)
        o0 = pl.multiple_of(cid * N + r0, 8)

        pltpu.sync_copy(z_hbm.at[pl.ds(r0, ROWS_PER)],
                        aggsh.at[pl.ds(r0, ROWS_PER)])
        plsc.subcore_barrier()

        rows = (rows0, rows1, rows2, rows3)
        gsems = (gsem0, gsem1, gsem2, gsem3)
        ssems = (ssem0, ssem1, ssem2, ssem3)

        lane = lax.iota(jnp.int32, 16)
        himask = lane >= 8

        def hist1d(ref1d, n):
            for j in range(n // 16):
                idxv = ref1d[pl.ds(16 * j, 16)]
                plsc.addupdate_scatter(cnt, [idxv], ones16)
            if n % 16:
                idxv = ref1d[pl.ds(n - 16, 16)]
                plsc.addupdate_scatter(cnt, [idxv], ones16, mask=himask)

        def oct_(t, carry):
            c0 = pl.multiple_of(8 * t, 8)
            pltpu.sync_copy(dst_hbm.at[w, pl.ds(c0, 8)], dstb)
            pltpu.sync_copy(src_hbm.at[w, pl.ds(c0, 8)], srcb)
            gs = [pltpu.async_copy(x_hbm.at[dstb.at[i]], rows[i],
                                   gsems[i]) for i in range(4)]
            base = pl.multiple_of(w * EPW + c0 * K, 8)
            pltpu.sync_copy(srcf_hbm.at[pl.ds(base, 8 * K)], srcl)
            hist1d(srcl, 8 * K)
            ss = []
            for i in range(4):
                gs[i].wait()
                ss.append(pltpu.async_copy(rows[i], aggsh.at[srcb.at[i]],
                                           ssems[i], add=True))
            gs2, ss2 = [], []
            for i in range(4):
                ss[i].wait()
                gs2.append(pltpu.async_copy(x_hbm.at[dstb.at[4 + i]], rows[i],
                                            gsems[i]))
            for i in range(4):
                gs2[i].wait()
                ss2.append(pltpu.async_copy(rows[i], aggsh.at[srcb.at[4 + i]],
                                            ssems[i], add=True))
            for s in ss2:
                s.wait()
            return carry

        OCTS = STEPS // 8
        lax.fori_loop(0, OCTS, oct_, 0)
        for c in range(8 * OCTS, STEPS):
            base = pl.multiple_of(w * EPW + c * K, 8)
            pltpu.sync_copy(dstf_hbm.at[pl.ds(base, K)], dsti1)
            pltpu.sync_copy(srcf_hbm.at[pl.ds(base, K)], srci1)
            pltpu.async_copy(x_hbm.at[dsti1], rows0, gsem0).wait()
            hist1d(srci1, K)
            pltpu.sync_copy(rows0, aggsh.at[srci1], add=True)
        plsc.subcore_barrier()

        pltpu.sync_copy(aggsh.at[pl.ds(r0, ROWS_PER)],
                        agg_out.at[pl.ds(o0, ROWS_PER)])
        s0 = pl.multiple_of(sid * N, 8)
        pltpu.sync_copy(cnt, cntsh.at[pl.ds(s0, N)])
        plsc.subcore_barrier()

        def zero_acc(i, carry):
            acc[pl.ds(i * 16, 16)] = z16
            return carry

        lax.fori_loop(0, ROWS_PER // 16, zero_acc, 0)

        def combine(r, carry):
            pltpu.sync_copy(cntsh.at[pl.ds(pl.multiple_of(r * N, 8) + r0,
                                           ROWS_PER)], tmp)

            def addv(i, c2):
                acc[pl.ds(i * 16, 16)] = (acc[pl.ds(i * 16, 16)]
                                          + tmp[pl.ds(i * 16, 16)])
                return c2

            lax.fori_loop(0, ROWS_PER // 16, addv, 0)
            return carry

        lax.fori_loop(0, _NS, combine, 0)
        pltpu.sync_copy(acc, deg_out.at[pl.ds(o0, ROWS_PER)])

    return sc_agg


@functools.lru_cache(maxsize=None)
def _make_tc_dense(N, C, D):
    R = 1000
    assert N % R == 0
    G = N // R

    def body(x_ref, a_ref, d_ref, ws_ref, bs_ref, wn_ref, bn_ref, o_ref):
        x = x_ref[...]
        agg = a_ref[0] + a_ref[1]
        deg = d_ref[0] + d_ref[1] + 1.0
        y = (agg + x) * (1.0 / deg)
        h = (jnp.dot(x, ws_ref[...], preferred_element_type=jnp.float32)
             + bs_ref[...]
             + jnp.dot(y, wn_ref[...], preferred_element_type=jnp.float32)
             + bn_ref[...])
        o_ref[...] = 0.5 * h * (1.0 + lax.erf(h * 0.7071067811865476))

    return pl.pallas_call(
        body,
        grid=(G,),
        in_specs=[
            pl.BlockSpec((R, C), lambda i: (i, 0)),
            pl.BlockSpec((_NC, R, C), lambda i: (0, i, 0)),
            pl.BlockSpec((_NC, R, 1), lambda i: (0, i, 0)),
            pl.BlockSpec((C, D), lambda i: (0, 0)),
            pl.BlockSpec((1, D), lambda i: (0, 0)),
            pl.BlockSpec((C, D), lambda i: (0, 0)),
            pl.BlockSpec((1, D), lambda i: (0, 0)),
        ],
        out_specs=pl.BlockSpec((R, D), lambda i: (i, 0)),
        out_shape=jax.ShapeDtypeStruct((N, D), jnp.float32),
    )


def kernel(x, edge_index, W_self, b_self, W_neigh, b_neigh):
    B, N, C = x.shape
    D = W_self.shape[1]
    E = edge_index.shape[1]
    x2d = x.reshape(N, C)
    NW = _NC * _NS
    steps = E // NW // _K
    src = edge_index[0].reshape(NW, steps, _K)
    dst = edge_index[1].reshape(NW, steps, _K)
    z = jnp.zeros((N, C), jnp.float32)

    agg_p, deg_p = _make_sc_agg(N, C, E)(
        src, dst, edge_index[0], edge_index[1], x2d, z)
    out = _make_tc_dense(N, C, D)(
        x2d, agg_p.reshape(_NC, N, C), deg_p.reshape(_NC, N, 1),
        W_self, b_self.reshape(1, D), W_neigh, b_neigh.reshape(1, D))
    return out.reshape(B, N, D)


# speedup vs baseline: 9.9579x; 1.1688x over previous
import functools

import jax
import jax.numpy as jnp
from jax import lax
from jax.experimental import pallas as pl
from jax.experimental.pallas import tpu as pltpu
from jax.experimental.pallas import tpu_sc as plsc

_NC = 2
_NS = 16
_K = 40


@functools.lru_cache(maxsize=None)
def _make_sc_agg(N, C, E):
    NW = _NC * _NS
    EPW = E // NW
    assert E % NW == 0
    K = _K
    assert EPW % K == 0 and K % 8 == 0
    STEPS = EPW // K
    ROWS_PER = 640
    STRIDE = 632
    assert (_NS - 1) * STRIDE + ROWS_PER >= N and N % 8 == 0 and N % 16 == 0

    mesh = plsc.VectorSubcoreMesh(core_axis_name="c", subcore_axis_name="s")

    @functools.partial(
        pl.kernel,
        mesh=mesh,
        compiler_params=pltpu.CompilerParams(needs_layout_passes=False),
        out_type=[
            jax.ShapeDtypeStruct((_NC * N, C), jnp.float32),
            jax.ShapeDtypeStruct((_NC * N,), jnp.float32),
        ],
        scratch_types=[
            pltpu.VMEM((8, K), jnp.int32),
            pltpu.VMEM((8, K), jnp.int32),
            pltpu.VMEM((8, K), jnp.int32),
            pltpu.VMEM((8, K), jnp.int32),
            pltpu.VMEM((K,), jnp.int32),
            pltpu.VMEM((K,), jnp.int32),
            pltpu.VMEM((8 * K,), jnp.int32),
            pltpu.VMEM((8 * K,), jnp.int32),
            pltpu.VMEM((K, C), jnp.float32),
            pltpu.VMEM((K, C), jnp.float32),
            pltpu.VMEM((K, C), jnp.float32),
            pltpu.VMEM((K, C), jnp.float32),
            pltpu.VMEM((N,), jnp.float32),
            pltpu.VMEM((ROWS_PER,), jnp.float32),
            pltpu.VMEM((ROWS_PER,), jnp.float32),
            pltpu.VMEM_SHARED((N, C), jnp.float32),
            pltpu.VMEM_SHARED((_NS * N,), jnp.float32),
        ] + [pltpu.SemaphoreType.DMA] * 14,
    )
    def sc_agg(src_hbm, dst_hbm, srcf_hbm, dstf_hbm, x_hbm, z_hbm,
               agg_out, deg_out,
               dstbP, srcbP, dstbQ, srcbQ, dsti1, srci1, srclP, srclQ,
               rows0, rows1, rows2, rows3,
               cnt, acc, tmp, aggsh, cntsh,
               gsem0, gsem1, gsem2, gsem3, ssem0, ssem1, ssem2, ssem3,
               ip0, ip1, ip2, iq0, iq1, iq2):
        cid = lax.axis_index("c")
        sid = lax.axis_index("s")
        w = cid * _NS + sid

        ones16 = jnp.full((16,), 1.0, dtype=jnp.float32)
        z16 = jnp.zeros((16,), jnp.float32)

        def zero_cnt(i, carry):
            cnt[pl.ds(i * 16, 16)] = z16
            return carry

        lax.fori_loop(0, N // 16, zero_cnt, 0)

        r0 = pl.multiple_of(jnp.minimum(sid * STRIDE, N - ROWS_PER), 8)
        o0 = pl.multiple_of(cid * N + r0, 8)

        pltpu.sync_copy(z_hbm.at[pl.ds(r0, ROWS_PER)],
                        aggsh.at[pl.ds(r0, ROWS_PER)])
        plsc.subcore_barrier()

        rows = (rows0, rows1, rows2, rows3)
        gsems = (gsem0, gsem1, gsem2, gsem3)
        ssems = (ssem0, ssem1, ssem2, ssem3)

        lane = lax.iota(jnp.int32, 16)
        himask = lane >= 8

        def hist1d(ref1d, n):
            for j in range(n // 16):
                idxv = ref1d[pl.ds(16 * j, 16)]
                plsc.addupdate_scatter(cnt, [idxv], ones16)
            if n % 16:
                idxv = ref1d[pl.ds(n - 16, 16)]
                plsc.addupdate_scatter(cnt, [idxv], ones16, mask=himask)

        setP = (dstbP, srcbP, srclP, (ip0, ip1, ip2))
        setQ = (dstbQ, srcbQ, srclQ, (iq0, iq1, iq2))

        def idx_descs(bufs, t):
            dstb, srcb, srcl, isems = bufs
            c0 = pl.multiple_of(8 * t, 8)
            base = pl.multiple_of(w * EPW + c0 * K, 8)
            return (
                (dst_hbm.at[w, pl.ds(c0, 8)], dstb, isems[0]),
                (src_hbm.at[w, pl.ds(c0, 8)], srcb, isems[1]),
                (srcf_hbm.at[pl.ds(base, 8 * K)], srcl, isems[2]),
            )

        def issue_idx(bufs, t):
            for sdm in idx_descs(bufs, t):
                pltpu.async_copy(*sdm)

        def wait_idx(bufs, t):
            for sdm in idx_descs(bufs, t):
                pltpu.make_async_copy(*sdm).wait()

        def run_oct(bufs, t):
            dstb, srcb, srcl, _ = bufs
            gs = [pltpu.async_copy(x_hbm.at[dstb.at[i]], rows[i],
                                   gsems[i]) for i in range(4)]
            hist1d(srcl, 8 * K)
            ss = []
            for i in range(4):
                gs[i].wait()
                ss.append(pltpu.async_copy(rows[i], aggsh.at[srcb.at[i]],
                                           ssems[i], add=True))
            gs2, ss2 = [], []
            for i in range(4):
                ss[i].wait()
                gs2.append(pltpu.async_copy(x_hbm.at[dstb.at[4 + i]], rows[i],
                                            gsems[i]))
            for i in range(4):
                gs2[i].wait()
                ss2.append(pltpu.async_copy(rows[i], aggsh.at[srcb.at[4 + i]],
                                            ssems[i], add=True))
            for s in ss2:
                s.wait()

        OCTS = STEPS // 8
        assert OCTS % 2 == 1 and OCTS >= 3
        issue_idx(setP, 0)

        def doct(i, carry):
            tA = 2 * i
            wait_idx(setP, tA)
            issue_idx(setQ, tA + 1)
            run_oct(setP, tA)
            wait_idx(setQ, tA + 1)
            issue_idx(setP, tA + 2)
            run_oct(setQ, tA + 1)
            return carry

        lax.fori_loop(0, OCTS // 2, doct, 0)
        wait_idx(setP, OCTS - 1)
        run_oct(setP, OCTS - 1)
        for c in range(8 * OCTS, STEPS):
            base = pl.multiple_of(w * EPW + c * K, 8)
            pltpu.sync_copy(dstf_hbm.at[pl.ds(base, K)], dsti1)
            pltpu.sync_copy(srcf_hbm.at[pl.ds(base, K)], srci1)
            pltpu.async_copy(x_hbm.at[dsti1], rows0, gsem0).wait()
            hist1d(srci1, K)
            pltpu.sync_copy(rows0, aggsh.at[srci1], add=True)
        plsc.subcore_barrier()

        pltpu.sync_copy(aggsh.at[pl.ds(r0, ROWS_PER)],
                        agg_out.at[pl.ds(o0, ROWS_PER)])
        s0 = pl.multiple_of(sid * N, 8)
        pltpu.sync_copy(cnt, cntsh.at[pl.ds(s0, N)])
        plsc.subcore_barrier()

        def zero_acc(i, carry):
            acc[pl.ds(i * 16, 16)] = z16
            return carry

        lax.fori_loop(0, ROWS_PER // 16, zero_acc, 0)

        def combine(r, carry):
            pltpu.sync_copy(cntsh.at[pl.ds(pl.multiple_of(r * N, 8) + r0,
                                           ROWS_PER)], tmp)

            def addv(i, c2):
                acc[pl.ds(i * 16, 16)] = (acc[pl.ds(i * 16, 16)]
                                          + tmp[pl.ds(i * 16, 16)])
                return c2

            lax.fori_loop(0, ROWS_PER // 16, addv, 0)
            return carry

        lax.fori_loop(0, _NS, combine, 0)
        pltpu.sync_copy(acc, deg_out.at[pl.ds(o0, ROWS_PER)])

    return sc_agg


@functools.lru_cache(maxsize=None)
def _make_tc_dense(N, C, D):
    R = 1000
    assert N % R == 0
    G = N // R

    def body(x_ref, a_ref, d_ref, ws_ref, bs_ref, wn_ref, bn_ref, o_ref):
        x = x_ref[...]
        agg = a_ref[0] + a_ref[1]
        deg = d_ref[0] + d_ref[1] + 1.0
        y = (agg + x) * (1.0 / deg)
        h = (jnp.dot(x, ws_ref[...], preferred_element_type=jnp.float32)
             + bs_ref[...]
             + jnp.dot(y, wn_ref[...], preferred_element_type=jnp.float32)
             + bn_ref[...])
        o_ref[...] = 0.5 * h * (1.0 + lax.erf(h * 0.7071067811865476))

    return pl.pallas_call(
        body,
        grid=(G,),
        in_specs=[
            pl.BlockSpec((R, C), lambda i: (i, 0)),
            pl.BlockSpec((_NC, R, C), lambda i: (0, i, 0)),
            pl.BlockSpec((_NC, R, 1), lambda i: (0, i, 0)),
            pl.BlockSpec((C, D), lambda i: (0, 0)),
            pl.BlockSpec((1, D), lambda i: (0, 0)),
            pl.BlockSpec((C, D), lambda i: (0, 0)),
            pl.BlockSpec((1, D), lambda i: (0, 0)),
        ],
        out_specs=pl.BlockSpec((R, D), lambda i: (i, 0)),
        out_shape=jax.ShapeDtypeStruct((N, D), jnp.float32),
    )


def kernel(x, edge_index, W_self, b_self, W_neigh, b_neigh):
    B, N, C = x.shape
    D = W_self.shape[1]
    E = edge_index.shape[1]
    x2d = x.reshape(N, C)
    NW = _NC * _NS
    steps = E // NW // _K
    src = edge_index[0].reshape(NW, steps, _K)
    dst = edge_index[1].reshape(NW, steps, _K)
    z = jnp.zeros((N, C), jnp.float32)

    agg_p, deg_p = _make_sc_agg(N, C, E)(
        src, dst, edge_index[0], edge_index[1], x2d, z)
    out = _make_tc_dense(N, C, D)(
        x2d, agg_p.reshape(_NC, N, C), deg_p.reshape(_NC, N, 1),
        W_self, b_self.reshape(1, D), W_neigh, b_neigh.reshape(1, D))
    return out.reshape(B, N, D)


# speedup vs baseline: 9.9760x; 1.0018x over previous
import functools

import jax
import jax.numpy as jnp
from jax import lax
from jax.experimental import pallas as pl
from jax.experimental.pallas import tpu as pltpu
from jax.experimental.pallas import tpu_sc as plsc

_NC = 2
_NS = 16
_K = 40


@functools.lru_cache(maxsize=None)
def _make_sc_agg(N, C, E):
    NW = _NC * _NS
    EPW = E // NW
    assert E % NW == 0
    K = _K
    assert EPW % K == 0 and K % 8 == 0
    STEPS = EPW // K
    ROWS_PER = 640
    STRIDE = 632
    assert (_NS - 1) * STRIDE + ROWS_PER >= N and N % 8 == 0 and N % 16 == 0

    mesh = plsc.VectorSubcoreMesh(core_axis_name="c", subcore_axis_name="s")

    @functools.partial(
        pl.kernel,
        mesh=mesh,
        compiler_params=pltpu.CompilerParams(needs_layout_passes=False),
        out_type=[
            jax.ShapeDtypeStruct((_NC * N, C), jnp.float32),
            jax.ShapeDtypeStruct((_NC * N,), jnp.float32),
        ],
        scratch_types=[
            pltpu.VMEM((8, K), jnp.int32),
            pltpu.VMEM((8, K), jnp.int32),
            pltpu.VMEM((8, K), jnp.int32),
            pltpu.VMEM((8, K), jnp.int32),
            pltpu.VMEM((K,), jnp.int32),
            pltpu.VMEM((K,), jnp.int32),
            pltpu.VMEM((8 * K,), jnp.int32),
            pltpu.VMEM((8 * K,), jnp.int32),
            pltpu.VMEM((K, C), jnp.float32),
            pltpu.VMEM((K, C), jnp.float32),
            pltpu.VMEM((K, C), jnp.float32),
            pltpu.VMEM((K, C), jnp.float32),
            pltpu.VMEM((N,), jnp.float32),
            pltpu.VMEM((ROWS_PER,), jnp.float32),
            pltpu.VMEM((ROWS_PER,), jnp.float32),
            pltpu.VMEM_SHARED((N, C), jnp.float32),
            pltpu.VMEM_SHARED((_NS * N,), jnp.float32),
        ] + [pltpu.SemaphoreType.DMA] * 14,
    )
    def sc_agg(src_hbm, dst_hbm, srcf_hbm, dstf_hbm, x_hbm, z_hbm,
               agg_out, deg_out,
               dstbP, srcbP, dstbQ, srcbQ, dsti1, srci1, srclP, srclQ,
               rows0, rows1, rows2, rows3,
               cnt, acc, tmp, aggsh, cntsh,
               gsem0, gsem1, gsem2, gsem3, ssem0, ssem1, ssem2, ssem3,
               ip0, ip1, ip2, iq0, iq1, iq2):
        cid = lax.axis_index("c")
        sid = lax.axis_index("s")
        w = cid * _NS + sid

        ones16 = jnp.full((16,), 1.0, dtype=jnp.float32)
        z16 = jnp.zeros((16,), jnp.float32)

        def zero_cnt(i, carry):
            cnt[pl.ds(i * 16, 16)] = z16
            return carry

        lax.fori_loop(0, N // 16, zero_cnt, 0)

        r0 = pl.multiple_of(jnp.minimum(sid * STRIDE, N - ROWS_PER), 8)
        o0 = pl.multiple_of(cid * N + r0, 8)

        pltpu.sync_copy(z_hbm.at[pl.ds(r0, ROWS_PER)],
                        aggsh.at[pl.ds(r0, ROWS_PER)])
        plsc.subcore_barrier()

        rows = (rows0, rows1, rows2, rows3)
        gsems = (gsem0, gsem1, gsem2, gsem3)
        ssems = (ssem0, ssem1, ssem2, ssem3)

        lane = lax.iota(jnp.int32, 16)
        himask = lane >= 8

        def hist1d(ref1d, n):
            for j in range(n // 16):
                idxv = ref1d[pl.ds(16 * j, 16)]
                plsc.addupdate_scatter(cnt, [idxv], ones16)
            if n % 16:
                idxv = ref1d[pl.ds(n - 16, 16)]
                plsc.addupdate_scatter(cnt, [idxv], ones16, mask=himask)

        setP = (dstbP, srcbP, srclP, (ip0, ip1, ip2))
        setQ = (dstbQ, srcbQ, srclQ, (iq0, iq1, iq2))

        def idx_descs(bufs, t):
            dstb, srcb, srcl, isems = bufs
            c0 = pl.multiple_of(8 * t, 8)
            base = pl.multiple_of(w * EPW + c0 * K, 8)
            return (
                (dst_hbm.at[w, pl.ds(c0, 8)], dstb, isems[0]),
                (src_hbm.at[w, pl.ds(c0, 8)], srcb, isems[1]),
                (srcf_hbm.at[pl.ds(base, 8 * K)], srcl, isems[2]),
            )

        def issue_idx(bufs, t):
            for sdm in idx_descs(bufs, t):
                pltpu.async_copy(*sdm)

        def wait_idx(bufs, t):
            for sdm in idx_descs(bufs, t):
                pltpu.make_async_copy(*sdm).wait()

        def drain_tail(bufs):
            _, srcb, _, _ = bufs
            for i in range(4):
                pltpu.make_async_copy(rows[i], aggsh.at[srcb.at[4 + i]],
                                      ssems[i]).wait()

        def run_oct(bufs, t):
            dstb, srcb, srcl, _ = bufs
            gs = [pltpu.async_copy(x_hbm.at[dstb.at[i]], rows[i],
                                   gsems[i]) for i in range(4)]
            hist1d(srcl, 8 * K)
            ss = []
            for i in range(4):
                gs[i].wait()
                ss.append(pltpu.async_copy(rows[i], aggsh.at[srcb.at[i]],
                                           ssems[i], add=True))
            gs2 = []
            for i in range(4):
                ss[i].wait()
                gs2.append(pltpu.async_copy(x_hbm.at[dstb.at[4 + i]], rows[i],
                                            gsems[i]))
            for i in range(4):
                gs2[i].wait()
                pltpu.async_copy(rows[i], aggsh.at[srcb.at[4 + i]],
                                 ssems[i], add=True)

        OCTS = STEPS // 8
        assert OCTS % 2 == 1 and OCTS >= 3
        issue_idx(setP, 0)
        wait_idx(setP, 0)
        issue_idx(setQ, 1)
        run_oct(setP, 0)

        def doct(i, carry):
            tB = 2 * i + 1
            wait_idx(setQ, tB)
            drain_tail(setP)
            issue_idx(setP, tB + 1)
            run_oct(setQ, tB)
            wait_idx(setP, tB + 1)
            drain_tail(setQ)
            nxt = jnp.minimum(tB + 2, OCTS - 1)
            issue_idx(setQ, nxt)
            run_oct(setP, tB + 1)
            return carry

        lax.fori_loop(0, OCTS // 2, doct, 0)
        drain_tail(setP)
        wait_idx(setQ, OCTS - 1)
        for c in range(8 * OCTS, STEPS):
            base = pl.multiple_of(w * EPW + c * K, 8)
            pltpu.sync_copy(dstf_hbm.at[pl.ds(base, K)], dsti1)
            pltpu.sync_copy(srcf_hbm.at[pl.ds(base, K)], srci1)
            pltpu.async_copy(x_hbm.at[dsti1], rows0, gsem0).wait()
            hist1d(srci1, K)
            pltpu.sync_copy(rows0, aggsh.at[srci1], add=True)
        plsc.subcore_barrier()

        pltpu.sync_copy(aggsh.at[pl.ds(r0, ROWS_PER)],
                        agg_out.at[pl.ds(o0, ROWS_PER)])
        s0 = pl.multiple_of(sid * N, 8)
        pltpu.sync_copy(cnt, cntsh.at[pl.ds(s0, N)])
        plsc.subcore_barrier()

        def zero_acc(i, carry):
            acc[pl.ds(i * 16, 16)] = z16
            return carry

        lax.fori_loop(0, ROWS_PER // 16, zero_acc, 0)

        def combine(r, carry):
            pltpu.sync_copy(cntsh.at[pl.ds(pl.multiple_of(r * N, 8) + r0,
                                           ROWS_PER)], tmp)

            def addv(i, c2):
                acc[pl.ds(i * 16, 16)] = (acc[pl.ds(i * 16, 16)]
                                          + tmp[pl.ds(i * 16, 16)])
                return c2

            lax.fori_loop(0, ROWS_PER // 16, addv, 0)
            return carry

        lax.fori_loop(0, _NS, combine, 0)
        pltpu.sync_copy(acc, deg_out.at[pl.ds(o0, ROWS_PER)])

    return sc_agg


@functools.lru_cache(maxsize=None)
def _make_tc_dense(N, C, D):
    R = 1000
    assert N % R == 0
    G = N // R

    def body(x_ref, a_ref, d_ref, ws_ref, bs_ref, wn_ref, bn_ref, o_ref):
        x = x_ref[...]
        agg = a_ref[0] + a_ref[1]
        deg = d_ref[0] + d_ref[1] + 1.0
        y = (agg + x) * (1.0 / deg)
        h = (jnp.dot(x, ws_ref[...], preferred_element_type=jnp.float32)
             + bs_ref[...]
             + jnp.dot(y, wn_ref[...], preferred_element_type=jnp.float32)
             + bn_ref[...])
        o_ref[...] = 0.5 * h * (1.0 + lax.erf(h * 0.7071067811865476))

    return pl.pallas_call(
        body,
        grid=(G,),
        in_specs=[
            pl.BlockSpec((R, C), lambda i: (i, 0)),
            pl.BlockSpec((_NC, R, C), lambda i: (0, i, 0)),
            pl.BlockSpec((_NC, R, 1), lambda i: (0, i, 0)),
            pl.BlockSpec((C, D), lambda i: (0, 0)),
            pl.BlockSpec((1, D), lambda i: (0, 0)),
            pl.BlockSpec((C, D), lambda i: (0, 0)),
            pl.BlockSpec((1, D), lambda i: (0, 0)),
        ],
        out_specs=pl.BlockSpec((R, D), lambda i: (i, 0)),
        out_shape=jax.ShapeDtypeStruct((N, D), jnp.float32),
    )


def kernel(x, edge_index, W_self, b_self, W_neigh, b_neigh):
    B, N, C = x.shape
    D = W_self.shape[1]
    E = edge_index.shape[1]
    x2d = x.reshape(N, C)
    NW = _NC * _NS
    steps = E // NW // _K
    src = edge_index[0].reshape(NW, steps, _K)
    dst = edge_index[1].reshape(NW, steps, _K)
    z = jnp.zeros((N, C), jnp.float32)

    agg_p, deg_p = _make_sc_agg(N, C, E)(
        src, dst, edge_index[0], edge_index[1], x2d, z)
    out = _make_tc_dense(N, C, D)(
        x2d, agg_p.reshape(_NC, N, C), deg_p.reshape(_NC, N, 1),
        W_self, b_self.reshape(1, D), W_neigh, b_neigh.reshape(1, D))
    return out.reshape(B, N, D)


# speedup vs baseline: 10.0251x; 1.0049x over previous
import functools

import jax
import jax.numpy as jnp
from jax import lax
from jax.experimental import pallas as pl
from jax.experimental.pallas import tpu as pltpu
from jax.experimental.pallas import tpu_sc as plsc

_NC = 2
_NS = 16
_K = 40


@functools.lru_cache(maxsize=None)
def _make_sc_agg(N, C, E):
    NW = _NC * _NS
    EPW = E // NW
    assert E % NW == 0
    K = _K
    assert EPW % K == 0 and K % 8 == 0
    STEPS = EPW // K
    ROWS_PER = 640
    STRIDE = 632
    assert (_NS - 1) * STRIDE + ROWS_PER >= N and N % 8 == 0 and N % 16 == 0

    mesh = plsc.VectorSubcoreMesh(core_axis_name="c", subcore_axis_name="s")

    @functools.partial(
        pl.kernel,
        mesh=mesh,
        compiler_params=pltpu.CompilerParams(needs_layout_passes=False),
        out_type=[
            jax.ShapeDtypeStruct((_NC * N, C), jnp.float32),
            jax.ShapeDtypeStruct((_NC * N,), jnp.float32),
        ],
        scratch_types=[
            pltpu.VMEM((8, K), jnp.int32),
            pltpu.VMEM((8, K), jnp.int32),
            pltpu.VMEM((8, K), jnp.int32),
            pltpu.VMEM((8, K), jnp.int32),
            pltpu.VMEM((K,), jnp.int32),
            pltpu.VMEM((K,), jnp.int32),
            pltpu.VMEM((8 * K,), jnp.int32),
            pltpu.VMEM((8 * K,), jnp.int32),
            pltpu.VMEM((K, C), jnp.float32),
            pltpu.VMEM((K, C), jnp.float32),
            pltpu.VMEM((K, C), jnp.float32),
            pltpu.VMEM((K, C), jnp.float32),
            pltpu.VMEM((N,), jnp.float32),
            pltpu.VMEM((ROWS_PER,), jnp.float32),
            pltpu.VMEM((ROWS_PER,), jnp.float32),
            pltpu.VMEM_SHARED((N, C), jnp.float32),
            pltpu.VMEM_SHARED((_NS * N,), jnp.float32),
        ] + [pltpu.SemaphoreType.DMA] * 14,
    )
    def sc_agg(src_hbm, dst_hbm, srcf_hbm, dstf_hbm, x_hbm, z_hbm,
               agg_out, deg_out,
               dstbP, srcbP, dstbQ, srcbQ, dsti1, srci1, srclP, srclQ,
               rows0, rows1, rows2, rows3,
               cnt, acc, tmp, aggsh, cntsh,
               gsem0, gsem1, gsem2, gsem3, ssem0, ssem1, ssem2, ssem3,
               ip0, ip1, ip2, iq0, iq1, iq2):
        cid = lax.axis_index("c")
        sid = lax.axis_index("s")
        w = cid * _NS + sid

        pltpu.async_copy(dst_hbm.at[w, pl.ds(0, 8)], dstbP, ip0)
        pltpu.async_copy(src_hbm.at[w, pl.ds(0, 8)], srcbP, ip1)
        pltpu.async_copy(
            srcf_hbm.at[pl.ds(pl.multiple_of(w * EPW, 8), 8 * K)], srclP, ip2)

        ones16 = jnp.full((16,), 1.0, dtype=jnp.float32)
        z16 = jnp.zeros((16,), jnp.float32)

        def zero_cnt(i, carry):
            cnt[pl.ds(i * 16, 16)] = z16
            return carry

        lax.fori_loop(0, N // 16, zero_cnt, 0)

        r0 = pl.multiple_of(jnp.minimum(sid * STRIDE, N - ROWS_PER), 8)
        o0 = pl.multiple_of(cid * N + r0, 8)

        pltpu.sync_copy(z_hbm.at[pl.ds(r0, ROWS_PER)],
                        aggsh.at[pl.ds(r0, ROWS_PER)])
        plsc.subcore_barrier()

        rows = (rows0, rows1, rows2, rows3)
        gsems = (gsem0, gsem1, gsem2, gsem3)
        ssems = (ssem0, ssem1, ssem2, ssem3)

        lane = lax.iota(jnp.int32, 16)
        himask = lane >= 8

        def hist1d(ref1d, n):
            for j in range(n // 16):
                idxv = ref1d[pl.ds(16 * j, 16)]
                plsc.addupdate_scatter(cnt, [idxv], ones16)
            if n % 16:
                idxv = ref1d[pl.ds(n - 16, 16)]
                plsc.addupdate_scatter(cnt, [idxv], ones16, mask=himask)

        setP = (dstbP, srcbP, srclP, (ip0, ip1, ip2))
        setQ = (dstbQ, srcbQ, srclQ, (iq0, iq1, iq2))

        def idx_descs(bufs, t):
            dstb, srcb, srcl, isems = bufs
            c0 = pl.multiple_of(8 * t, 8)
            base = pl.multiple_of(w * EPW + c0 * K, 8)
            return (
                (dst_hbm.at[w, pl.ds(c0, 8)], dstb, isems[0]),
                (src_hbm.at[w, pl.ds(c0, 8)], srcb, isems[1]),
                (srcf_hbm.at[pl.ds(base, 8 * K)], srcl, isems[2]),
            )

        def issue_idx(bufs, t):
            for sdm in idx_descs(bufs, t):
                pltpu.async_copy(*sdm)

        def wait_idx(bufs, t):
            for sdm in idx_descs(bufs, t):
                pltpu.make_async_copy(*sdm).wait()

        def drain_tail(bufs):
            _, srcb, _, _ = bufs
            for i in range(4):
                pltpu.make_async_copy(rows[i], aggsh.at[srcb.at[4 + i]],
                                      ssems[i]).wait()

        def run_oct(bufs, t):
            dstb, srcb, srcl, _ = bufs
            gs = [pltpu.async_copy(x_hbm.at[dstb.at[i]], rows[i],
                                   gsems[i]) for i in range(4)]
            hist1d(srcl, 8 * K)
            ss = []
            for i in range(4):
                gs[i].wait()
                ss.append(pltpu.async_copy(rows[i], aggsh.at[srcb.at[i]],
                                           ssems[i], add=True))
            gs2 = []
            for i in range(4):
                ss[i].wait()
                gs2.append(pltpu.async_copy(x_hbm.at[dstb.at[4 + i]], rows[i],
                                            gsems[i]))
            for i in range(4):
                gs2[i].wait()
                pltpu.async_copy(rows[i], aggsh.at[srcb.at[4 + i]],
                                 ssems[i], add=True)

        OCTS = STEPS // 8
        assert OCTS % 2 == 1 and OCTS >= 3
        wait_idx(setP, 0)
        issue_idx(setQ, 1)
        run_oct(setP, 0)

        def doct(i, carry):
            tB = 2 * i + 1
            wait_idx(setQ, tB)
            drain_tail(setP)
            issue_idx(setP, tB + 1)
            run_oct(setQ, tB)
            wait_idx(setP, tB + 1)
            drain_tail(setQ)
            nxt = jnp.minimum(tB + 2, OCTS - 1)
            issue_idx(setQ, nxt)
            run_oct(setP, tB + 1)
            return carry

        lax.fori_loop(0, OCTS // 2, doct, 0)
        drain_tail(setP)
        wait_idx(setQ, OCTS - 1)
        for c in range(8 * OCTS, STEPS):
            base = pl.multiple_of(w * EPW + c * K, 8)
            pltpu.sync_copy(dstf_hbm.at[pl.ds(base, K)], dsti1)
            pltpu.sync_copy(srcf_hbm.at[pl.ds(base, K)], srci1)
            pltpu.async_copy(x_hbm.at[dsti1], rows0, gsem0).wait()
            hist1d(srci1, K)
            pltpu.sync_copy(rows0, aggsh.at[srci1], add=True)
        plsc.subcore_barrier()

        pltpu.sync_copy(aggsh.at[pl.ds(r0, ROWS_PER)],
                        agg_out.at[pl.ds(o0, ROWS_PER)])
        s0 = pl.multiple_of(sid * N, 8)
        pltpu.sync_copy(cnt, cntsh.at[pl.ds(s0, N)])
        plsc.subcore_barrier()

        def zero_acc(i, carry):
            acc[pl.ds(i * 16, 16)] = z16
            return carry

        lax.fori_loop(0, ROWS_PER // 16, zero_acc, 0)

        def combine(r, carry):
            pltpu.sync_copy(cntsh.at[pl.ds(pl.multiple_of(r * N, 8) + r0,
                                           ROWS_PER)], tmp)

            def addv(i, c2):
                acc[pl.ds(i * 16, 16)] = (acc[pl.ds(i * 16, 16)]
                                          + tmp[pl.ds(i * 16, 16)])
                return c2

            lax.fori_loop(0, ROWS_PER // 16, addv, 0)
            return carry

        lax.fori_loop(0, _NS, combine, 0)
        pltpu.sync_copy(acc, deg_out.at[pl.ds(o0, ROWS_PER)])

    return sc_agg


@functools.lru_cache(maxsize=None)
def _make_tc_dense(N, C, D):
    R = 1000
    assert N % R == 0
    G = N // R

    def body(x_ref, a_ref, d_ref, ws_ref, bs_ref, wn_ref, bn_ref, o_ref):
        x = x_ref[...]
        agg = a_ref[0] + a_ref[1]
        deg = d_ref[0] + d_ref[1] + 1.0
        y = (agg + x) * (1.0 / deg)
        h = (jnp.dot(x, ws_ref[...], preferred_element_type=jnp.float32)
             + bs_ref[...]
             + jnp.dot(y, wn_ref[...], preferred_element_type=jnp.float32)
             + bn_ref[...])
        o_ref[...] = 0.5 * h * (1.0 + lax.erf(h * 0.7071067811865476))

    return pl.pallas_call(
        body,
        grid=(G,),
        in_specs=[
            pl.BlockSpec((R, C), lambda i: (i, 0)),
            pl.BlockSpec((_NC, R, C), lambda i: (0, i, 0)),
            pl.BlockSpec((_NC, R, 1), lambda i: (0, i, 0)),
            pl.BlockSpec((C, D), lambda i: (0, 0)),
            pl.BlockSpec((1, D), lambda i: (0, 0)),
            pl.BlockSpec((C, D), lambda i: (0, 0)),
            pl.BlockSpec((1, D), lambda i: (0, 0)),
        ],
        out_specs=pl.BlockSpec((R, D), lambda i: (i, 0)),
        out_shape=jax.ShapeDtypeStruct((N, D), jnp.float32),
    )


def kernel(x, edge_index, W_self, b_self, W_neigh, b_neigh):
    B, N, C = x.shape
    D = W_self.shape[1]
    E = edge_index.shape[1]
    x2d = x.reshape(N, C)
    NW = _NC * _NS
    steps = E // NW // _K
    src = edge_index[0].reshape(NW, steps, _K)
    dst = edge_index[1].reshape(NW, steps, _K)
    z = jnp.zeros((N, C), jnp.float32)

    agg_p, deg_p = _make_sc_agg(N, C, E)(
        src, dst, edge_index[0], edge_index[1], x2d, z)
    out = _make_tc_dense(N, C, D)(
        x2d, agg_p.reshape(_NC, N, C), deg_p.reshape(_NC, N, 1),
        W_self, b_self.reshape(1, D), W_neigh, b_neigh.reshape(1, D))
    return out.reshape(B, N, D)
